# baseline (speedup 1.0000x reference)
import functools

import jax
import jax.numpy as jnp
from jax import lax
from jax.experimental import pallas as pl
from jax.experimental.pallas import tpu as pltpu
from jax.experimental.pallas import tpu_sc as plsc

B = 16384
D = 64
NC = 2
NS = 16
NW = NC * NS
BPW = B // NW
CHUNK = 128
NCH = BPW // CHUNK
NROWS = B // CHUNK

_sc_mesh = plsc.VectorSubcoreMesh(core_axis_name="c", subcore_axis_name="s")


@functools.partial(
    pl.kernel,
    out_type=[
        jax.ShapeDtypeStruct((NROWS, CHUNK, D), jnp.float32),
        jax.ShapeDtypeStruct((NROWS, CHUNK, D), jnp.float32),
    ],
    mesh=_sc_mesh,
    compiler_params=pltpu.CompilerParams(use_tc_tiling_on_sc=False),
    scratch_types=[
        pltpu.VMEM((NCH, CHUNK), jnp.int32),
        pltpu.VMEM((NCH, CHUNK), jnp.int32),
        pltpu.VMEM((NCH, CHUNK, D), jnp.float32),
        pltpu.VMEM((NCH, CHUNK, D), jnp.float32),
        pltpu.SemaphoreType.DMA,
    ],
)
def _sc_gather(uidx_hbm, bidx_hbm, utab_hbm, btab_hbm, uout, bout,
               uidx_v, bidx_v, urows_v, brows_v, sem):
    wid = lax.axis_index("s") * NC + lax.axis_index("c")
    row0 = wid * NCH
    pltpu.sync_copy(uidx_hbm.at[pl.ds(row0, NCH)], uidx_v)
    pltpu.sync_copy(bidx_hbm.at[pl.ds(row0, NCH)], bidx_v)
    copies = []
    for j in range(NCH):
        copies.append(pltpu.async_copy(utab_hbm.at[uidx_v.at[j]], urows_v.at[j], sem))
        copies.append(pltpu.async_copy(btab_hbm.at[bidx_v.at[j]], brows_v.at[j], sem))
    for c in copies:
        c.wait()
    pltpu.sync_copy(urows_v, uout.at[pl.ds(row0, NCH)])
    pltpu.sync_copy(brows_v, bout.at[pl.ds(row0, NCH)])


BLK = 2048


def _mlp_body(u_ref, b_ref, w1u_ref, w1b_ref, b1_ref, w2_ref, b2_ref,
              w3_ref, b3_ref, w4_ref, b4_ref, out_ref):
    h = (jnp.dot(u_ref[...], w1u_ref[...], preferred_element_type=jnp.float32)
         + jnp.dot(b_ref[...], w1b_ref[...], preferred_element_type=jnp.float32)
         + b1_ref[...])
    h = jnp.maximum(h, 0.0)
    h = jnp.dot(h, w2_ref[...], preferred_element_type=jnp.float32) + b2_ref[...]
    h = jnp.maximum(h, 0.0)
    h = jnp.dot(h, w3_ref[...], preferred_element_type=jnp.float32) + b3_ref[...]
    h = jnp.maximum(h, 0.0)
    o = jnp.dot(h, w4_ref[...], preferred_element_type=jnp.float32) + b4_ref[...]
    out_ref[...] = jax.nn.sigmoid(o)


_mlp_call = pl.pallas_call(
    _mlp_body,
    grid=(B // BLK,),
    in_specs=[
        pl.BlockSpec((BLK, D), lambda i: (i, 0)),
        pl.BlockSpec((BLK, D), lambda i: (i, 0)),
        pl.BlockSpec((D, 128), lambda i: (0, 0)),
        pl.BlockSpec((D, 128), lambda i: (0, 0)),
        pl.BlockSpec((1, 128), lambda i: (0, 0)),
        pl.BlockSpec((128, 64), lambda i: (0, 0)),
        pl.BlockSpec((1, 64), lambda i: (0, 0)),
        pl.BlockSpec((64, 32), lambda i: (0, 0)),
        pl.BlockSpec((1, 32), lambda i: (0, 0)),
        pl.BlockSpec((32, 1), lambda i: (0, 0)),
        pl.BlockSpec((1, 1), lambda i: (0, 0)),
    ],
    out_specs=pl.BlockSpec((BLK, 1), lambda i: (i, 0)),
    out_shape=jax.ShapeDtypeStruct((B, 1), jnp.float32),
)


def kernel(user, business, user_table, business_table, W1, b1, W2, b2, W3, b3, W4, b4):
    uidx = user.astype(jnp.int32).reshape(NROWS, CHUNK)
    bidx = business.astype(jnp.int32).reshape(NROWS, CHUNK)
    u3, b3d = _sc_gather(uidx, bidx, user_table, business_table)
    u_emb = u3.reshape(B, D)
    b_emb = b3d.reshape(B, D)
    w1ut = W1[:, :D].T
    w1bt = W1[:, D:].T
    out = _mlp_call(u_emb, b_emb, w1ut, w1bt, b1.reshape(1, 128),
                    W2.T, b2.reshape(1, 64), W3.T, b3.reshape(1, 32),
                    W4.T, b4.reshape(1, 1))
    return out[:, 0]


# speedup vs baseline: 2.1715x; 2.1715x over previous
import functools

import jax
import jax.numpy as jnp
from jax import lax
from jax.experimental import pallas as pl
from jax.experimental.pallas import tpu as pltpu
from jax.experimental.pallas import tpu_sc as plsc

B = 16384
D = 64
NC = 2
NS = 16
NW = NC * NS
EPT = B // NW
GB = 8
NG = EPT // GB

_sc_mesh = plsc.VectorSubcoreMesh(core_axis_name="c", subcore_axis_name="s")


@functools.partial(
    pl.kernel,
    out_type=[
        jax.ShapeDtypeStruct((B, D), jnp.float32),
        jax.ShapeDtypeStruct((B, D), jnp.float32),
    ],
    mesh=_sc_mesh,
    compiler_params=pltpu.CompilerParams(
        use_tc_tiling_on_sc=True, needs_layout_passes=False),
    scratch_types=[
        pltpu.VMEM((EPT + 16,), jnp.int32),
        pltpu.VMEM((EPT + 16,), jnp.int32),
        pltpu.VMEM((GB, D, 128), jnp.float32),
        pltpu.VMEM((GB, D), jnp.float32),
        pltpu.SemaphoreType.DMA,
    ],
)
def _sc_gather(uidx, bidx, utabT, btabT, uout, bout,
               uidx_v, bidx_v, buf, outv, gsem):
    wid = lax.axis_index("s") * NC + lax.axis_index("c")
    base = wid * EPT
    pltpu.sync_copy(uidx.at[pl.ds(base, EPT)], uidx_v.at[pl.ds(0, EPT)])
    pltpu.sync_copy(bidx.at[pl.ds(base, EPT)], bidx_v.at[pl.ds(0, EPT)])

    def run(idx_v, tabT, out_hbm):
        def scalar(vec, k):
            return lax.squeeze(lax.slice(vec, (k,), (k + 1,)), (0,))

        def body(g, carry):
            sv = idx_v[pl.ds(g * GB, 16)]
            starts = lax.bitwise_and(sv, jnp.int32(-128))
            lanes = lax.bitwise_and(sv, jnp.int32(127))
            for k in range(GB):
                start = pl.multiple_of(scalar(starts, k), 128)
                pltpu.async_copy(tabT.at[:, pl.ds(start, 128)], buf.at[k], gsem)
            for k in range(GB):
                pltpu.make_async_copy(
                    tabT.at[:, pl.ds(0, 128)], buf.at[k], gsem).wait()
            for k in range(GB):
                cols = jnp.full((16,), scalar(lanes, k), jnp.int32)
                for q in range(D // 16):
                    rows = lax.iota(jnp.int32, 16) + q * 16
                    vals = plsc.load_gather(buf.at[k], [rows, cols])
                    outv[k, pl.ds(q * 16, 16)] = vals
            pltpu.sync_copy(outv, out_hbm.at[pl.ds(base + g * GB, GB)])
            return carry

        lax.fori_loop(0, NG, body, 0)

    run(uidx_v, utabT, uout)
    run(bidx_v, btabT, bout)


BLK = 2048


def _mlp_body(u_ref, b_ref, w1u_ref, w1b_ref, b1_ref, w2_ref, b2_ref,
              w3_ref, b3_ref, w4_ref, b4_ref, out_ref):
    h = (jnp.dot(u_ref[...], w1u_ref[...], preferred_element_type=jnp.float32)
         + jnp.dot(b_ref[...], w1b_ref[...], preferred_element_type=jnp.float32)
         + b1_ref[...])
    h = jnp.maximum(h, 0.0)
    h = jnp.dot(h, w2_ref[...], preferred_element_type=jnp.float32) + b2_ref[...]
    h = jnp.maximum(h, 0.0)
    h = jnp.dot(h, w3_ref[...], preferred_element_type=jnp.float32) + b3_ref[...]
    h = jnp.maximum(h, 0.0)
    o = jnp.dot(h, w4_ref[...], preferred_element_type=jnp.float32) + b4_ref[...]
    out_ref[...] = jax.nn.sigmoid(o)


_mlp_call = pl.pallas_call(
    _mlp_body,
    grid=(B // BLK,),
    in_specs=[
        pl.BlockSpec((BLK, D), lambda i: (i, 0)),
        pl.BlockSpec((BLK, D), lambda i: (i, 0)),
        pl.BlockSpec((D, 128), lambda i: (0, 0)),
        pl.BlockSpec((D, 128), lambda i: (0, 0)),
        pl.BlockSpec((1, 128), lambda i: (0, 0)),
        pl.BlockSpec((128, 64), lambda i: (0, 0)),
        pl.BlockSpec((1, 64), lambda i: (0, 0)),
        pl.BlockSpec((64, 32), lambda i: (0, 0)),
        pl.BlockSpec((1, 32), lambda i: (0, 0)),
        pl.BlockSpec((32, 1), lambda i: (0, 0)),
        pl.BlockSpec((1, 1), lambda i: (0, 0)),
    ],
    out_specs=pl.BlockSpec((BLK, 1), lambda i: (i, 0)),
    out_shape=jax.ShapeDtypeStruct((B, 1), jnp.float32),
)


def kernel(user, business, user_table, business_table, W1, b1, W2, b2, W3, b3, W4, b4):
    uidx = user.astype(jnp.int32)
    bidx = business.astype(jnp.int32)
    u_emb, b_emb = _sc_gather(uidx, bidx, user_table.T, business_table.T)
    w1ut = W1[:, :D].T
    w1bt = W1[:, D:].T
    out = _mlp_call(u_emb, b_emb, w1ut, w1bt, b1.reshape(1, 128),
                    W2.T, b2.reshape(1, 64), W3.T, b3.reshape(1, 32),
                    W4.T, b4.reshape(1, 1))
    return out[:, 0]


# speedup vs baseline: 2.8891x; 1.3305x over previous
import functools

import jax
import jax.numpy as jnp
from jax import lax
from jax.experimental import pallas as pl
from jax.experimental.pallas import tpu as pltpu
from jax.experimental.pallas import tpu_sc as plsc

B = 16384
D = 64
NC = 2
NS = 16
NW = NC * NS
EPT = B // NW
GB = 4
NG = EPT // GB

_sc_mesh = plsc.VectorSubcoreMesh(core_axis_name="c", subcore_axis_name="s")


@functools.partial(
    pl.kernel,
    out_type=[
        jax.ShapeDtypeStruct((B, D), jnp.float32),
        jax.ShapeDtypeStruct((B, D), jnp.float32),
    ],
    mesh=_sc_mesh,
    compiler_params=pltpu.CompilerParams(
        use_tc_tiling_on_sc=True, needs_layout_passes=False),
    scratch_types=[
        pltpu.VMEM((EPT + 16,), jnp.int32),
        pltpu.VMEM((EPT + 16,), jnp.int32),
        pltpu.VMEM((2, GB, D, 128), jnp.float32),
        pltpu.VMEM((2, GB, D), jnp.float32),
        pltpu.SemaphoreType.DMA,
        pltpu.SemaphoreType.DMA,
    ],
)
def _sc_gather(uidx, bidx, utabT, btabT, uout, bout,
               uidx_v, bidx_v, buf, outv, gsem, osem):
    wid = lax.axis_index("s") * NC + lax.axis_index("c")
    base = wid * EPT
    pltpu.sync_copy(uidx.at[pl.ds(base, EPT)], uidx_v.at[pl.ds(0, EPT)])
    pltpu.sync_copy(bidx.at[pl.ds(base, EPT)], bidx_v.at[pl.ds(0, EPT)])

    def scalar(vec, k):
        return lax.squeeze(lax.slice(vec, (k,), (k + 1,)), (0,))

    def run(idx_v, tabT, out_hbm):
        def fire(g, slot):
            sv = idx_v[pl.ds(g * GB, 16)]
            starts = lax.bitwise_and(sv, jnp.int32(-128))
            for k in range(GB):
                start = pl.multiple_of(scalar(starts, k), 128)
                pltpu.async_copy(
                    tabT.at[:, pl.ds(start, 128)], buf.at[slot, k], gsem)

        fire(0, 0)

        def body(g, carry):
            slot = lax.rem(g, 2)

            @pl.when(g + 1 < NG)
            def _():
                fire(g + 1, 1 - slot)

            for k in range(GB):
                pltpu.make_async_copy(
                    tabT.at[:, pl.ds(0, 128)], buf.at[slot, k], gsem).wait()
            sv = idx_v[pl.ds(g * GB, 16)]
            lanes = lax.bitwise_and(sv, jnp.int32(127))
            for k in range(GB):
                cols = jnp.full((16,), scalar(lanes, k), jnp.int32)
                for q in range(D // 16):
                    rows = lax.iota(jnp.int32, 16) + q * 16
                    vals = plsc.load_gather(buf.at[slot, k], [rows, cols])
                    outv[slot, k, pl.ds(q * 16, 16)] = vals
            pltpu.async_copy(
                outv.at[slot], out_hbm.at[pl.ds(base + g * GB, GB)], osem)

            @pl.when(g >= 1)
            def _():
                pltpu.make_async_copy(
                    out_hbm.at[pl.ds(base, GB)], outv.at[1 - slot], osem).wait()

            return carry

        lax.fori_loop(0, NG, body, 0)
        pltpu.make_async_copy(
            out_hbm.at[pl.ds(base, GB)], outv.at[lax.rem(NG - 1, 2)],
            osem).wait()

    run(uidx_v, utabT, uout)
    run(bidx_v, btabT, bout)


BLK = 2048


def _mlp_body(u_ref, b_ref, w1u_ref, w1b_ref, b1_ref, w2_ref, b2_ref,
              w3_ref, b3_ref, w4_ref, b4_ref, out_ref):
    h = (jnp.dot(u_ref[...], w1u_ref[...], preferred_element_type=jnp.float32)
         + jnp.dot(b_ref[...], w1b_ref[...], preferred_element_type=jnp.float32)
         + b1_ref[...])
    h = jnp.maximum(h, 0.0)
    h = jnp.dot(h, w2_ref[...], preferred_element_type=jnp.float32) + b2_ref[...]
    h = jnp.maximum(h, 0.0)
    h = jnp.dot(h, w3_ref[...], preferred_element_type=jnp.float32) + b3_ref[...]
    h = jnp.maximum(h, 0.0)
    o = jnp.dot(h, w4_ref[...], preferred_element_type=jnp.float32) + b4_ref[...]
    out_ref[...] = jax.nn.sigmoid(o)


_mlp_call = pl.pallas_call(
    _mlp_body,
    grid=(B // BLK,),
    in_specs=[
        pl.BlockSpec((BLK, D), lambda i: (i, 0)),
        pl.BlockSpec((BLK, D), lambda i: (i, 0)),
        pl.BlockSpec((D, 128), lambda i: (0, 0)),
        pl.BlockSpec((D, 128), lambda i: (0, 0)),
        pl.BlockSpec((1, 128), lambda i: (0, 0)),
        pl.BlockSpec((128, 64), lambda i: (0, 0)),
        pl.BlockSpec((1, 64), lambda i: (0, 0)),
        pl.BlockSpec((64, 32), lambda i: (0, 0)),
        pl.BlockSpec((1, 32), lambda i: (0, 0)),
        pl.BlockSpec((32, 1), lambda i: (0, 0)),
        pl.BlockSpec((1, 1), lambda i: (0, 0)),
    ],
    out_specs=pl.BlockSpec((BLK, 1), lambda i: (i, 0)),
    out_shape=jax.ShapeDtypeStruct((B, 1), jnp.float32),
)


def kernel(user, business, user_table, business_table, W1, b1, W2, b2, W3, b3, W4, b4):
    uidx = user.astype(jnp.int32)
    bidx = business.astype(jnp.int32)
    u_emb, b_emb = _sc_gather(uidx, bidx, user_table.T, business_table.T)
    w1ut = W1[:, :D].T
    w1bt = W1[:, D:].T
    out = _mlp_call(u_emb, b_emb, w1ut, w1bt, b1.reshape(1, 128),
                    W2.T, b2.reshape(1, 64), W3.T, b3.reshape(1, 32),
                    W4.T, b4.reshape(1, 1))
    return out[:, 0]
